# Initial kernel scaffold; baseline (speedup 1.0000x reference)
import jax
import jax.numpy as jnp
from jax.experimental import pallas as pl


def kernel(x, tables):
    raise NotImplementedError("write your pallas kernel here")


# speedup vs baseline: 124.7890x; 124.7890x over previous
import functools

import jax
import jax.numpy as jnp
import numpy as np
from jax import lax
from jax.experimental import pallas as pl
from jax.experimental.pallas import tpu as pltpu
from jax.experimental.pallas import tpu_sc as plsc

_DIM = 3
_N_LEVELS = 16
_N_FEATS = 2
_TABLE_SIZE = 2 ** 15
_BASE_RES = 16
_FINEST_RES = 512
_RANGE = 1.0

_P1 = 2654435761
_P2 = 805459861

_NC = 2
_NS = 16
_NW = _NC * _NS

_C = 2048
_LANES = 16


def _resolutions_list():
    b = np.exp((np.log(_FINEST_RES) - np.log(_BASE_RES)) / (_N_LEVELS - 1))
    return [int(np.floor(_BASE_RES * (b ** l))) for l in range(_N_LEVELS)]


_RESS = _resolutions_list()


def _make_sc_kernel(npad):
    ppw = npad // _NW
    nchunk = ppw // _C
    nvec = _C // _LANES
    mesh = plsc.VectorSubcoreMesh(core_axis_name="c", subcore_axis_name="s")

    @functools.partial(
        pl.kernel,
        out_type=jax.ShapeDtypeStruct((_N_LEVELS, _N_FEATS, npad), jnp.float32),
        mesh=mesh,
        compiler_params=pltpu.CompilerParams(needs_layout_passes=False),
        scratch_types=[
            pltpu.VMEM((_TABLE_SIZE * _N_FEATS,), jnp.float32),
            pltpu.VMEM((_DIM, _C), jnp.float32),
            pltpu.VMEM((_N_FEATS, _C), jnp.float32),
        ],
    )
    def hashgrid_sc(x_hbm, tables_hbm, out_hbm, table_v, x_v, o_v):
        wid = lax.axis_index("s") * _NC + lax.axis_index("c")
        base = wid * ppw

        for l in range(_N_LEVELS):
            resf = float(_RESS[l])
            pltpu.sync_copy(tables_hbm.at[l], table_v)

            def chunk_body(c, carry, _resf=resf, _l=l):
                cb = base + c * _C
                pltpu.sync_copy(x_hbm.at[:, pl.ds(cb, _C)], x_v)

                def vstep(i, carry2):
                    off = i * _LANES
                    x0 = x_v[0, pl.ds(off, _LANES)]
                    x1 = x_v[1, pl.ds(off, _LANES)]
                    x2 = x_v[2, pl.ds(off, _LANES)]
                    xs0 = ((x0 + 1.0) * 0.5) * _resf
                    xs1 = ((x1 + 1.0) * 0.5) * _resf
                    xs2 = ((x2 + 1.0) * 0.5) * _resf
                    xi0 = xs0.astype(jnp.int32)
                    xi1 = xs1.astype(jnp.int32)
                    xi2 = xs2.astype(jnp.int32)
                    xf0 = xs0 - xi0.astype(jnp.float32)
                    xf1 = xs1 - xi1.astype(jnp.float32)
                    xf2 = xs2 - xi2.astype(jnp.float32)

                    c0a = xi0.astype(jnp.uint32)
                    c0b = c0a + jnp.uint32(1)
                    t1a = xi1.astype(jnp.uint32) * jnp.uint32(_P1)
                    t1b = t1a + jnp.uint32(_P1)
                    t2a = xi2.astype(jnp.uint32) * jnp.uint32(_P2)
                    t2b = t2a + jnp.uint32(_P2)

                    w0a = 1.0 - xf0
                    w0b = xf0
                    acc0 = jnp.zeros((_LANES,), jnp.float32)
                    acc1 = jnp.zeros((_LANES,), jnp.float32)
                    for (t1, w1) in ((t1a, 1.0 - xf1), (t1b, xf1)):
                        for (t2, w2) in ((t2a, 1.0 - xf2), (t2b, xf2)):
                            h12 = t1 ^ t2
                            w12 = w1 * w2
                            ia = (((c0a ^ h12) & jnp.uint32(0x7FFF))
                                  << jnp.uint32(1)).astype(jnp.int32)
                            ib = (((c0b ^ h12) & jnp.uint32(0x7FFF))
                                  << jnp.uint32(1)).astype(jnp.int32)
                            fa0 = plsc.load_gather(table_v, [ia])
                            fa1 = plsc.load_gather(table_v, [ia + 1])
                            fb0 = plsc.load_gather(table_v, [ib])
                            fb1 = plsc.load_gather(table_v, [ib + 1])
                            s0 = fa0 * w0a + fb0 * w0b
                            s1 = fa1 * w0a + fb1 * w0b
                            acc0 = acc0 + w12 * s0
                            acc1 = acc1 + w12 * s1
                    o_v[0, pl.ds(off, _LANES)] = acc0
                    o_v[1, pl.ds(off, _LANES)] = acc1
                    return carry2

                lax.fori_loop(0, nvec, vstep, 0)
                pltpu.sync_copy(o_v, out_hbm.at[_l, :, pl.ds(cb, _C)])
                return carry

            lax.fori_loop(0, nchunk, chunk_body, 0)

    return hashgrid_sc


def kernel(x, tables):
    n = x.shape[0]
    block = _NW * _C
    npad = ((n + block - 1) // block) * block
    x_t = jnp.transpose(x)
    if npad != n:
        x_t = jnp.pad(x_t, ((0, 0), (0, npad - n)))
    tab = tables.reshape(_N_LEVELS, _TABLE_SIZE * _N_FEATS)
    out = _make_sc_kernel(npad)(x_t, tab)
    enc = jnp.transpose(out, (2, 0, 1))[:n]
    return enc.reshape(n, _N_LEVELS * _N_FEATS)
